# Initial kernel scaffold; baseline (speedup 1.0000x reference)
import jax
import jax.numpy as jnp
from jax.experimental import pallas as pl


def kernel(input_ids, table):
    raise NotImplementedError("write your pallas kernel here")


# speedup vs baseline: 6.3554x; 6.3554x over previous
import functools

import jax
import jax.numpy as jnp
from jax import lax
from jax.experimental import pallas as pl
from jax.experimental.pallas import tpu as pltpu
from jax.experimental.pallas import tpu_sc as plsc


def kernel(input_ids, table):
    B0, S = input_ids.shape
    V, D = table.shape
    B = B0 * S

    info = plsc.get_sparse_core_info()
    NC, NS = info.num_cores, info.num_subcores
    NW = NC * NS

    CHUNK = 128
    b_per_w = B // NW
    n_chunks = b_per_w // CHUNK
    assert b_per_w * NW == B and n_chunks * CHUNK == b_per_w

    idx3 = input_ids.reshape(NW, n_chunks, CHUNK).astype(jnp.int32)

    mesh = plsc.VectorSubcoreMesh(core_axis_name="c", subcore_axis_name="s")

    @functools.partial(
        pl.kernel,
        mesh=mesh,
        out_type=jax.ShapeDtypeStruct((B, D), jnp.float32),
        scratch_types=[
            pltpu.VMEM((n_chunks, CHUNK), jnp.int32),
            pltpu.VMEM((CHUNK, D), jnp.float32),
            pltpu.SemaphoreType.DMA,
        ],
    )
    def emb(idx_hbm, table_hbm, out_hbm, idx_v, rows_v, sem):
        wid = lax.axis_index("s") * NC + lax.axis_index("c")
        base = wid * b_per_w
        pltpu.sync_copy(idx_hbm.at[wid], idx_v)

        def body(j, carry):
            pltpu.async_copy(table_hbm.at[idx_v.at[j]], rows_v, sem).wait()
            pltpu.sync_copy(rows_v, out_hbm.at[pl.ds(base + j * CHUNK, CHUNK)])
            return carry

        lax.fori_loop(0, n_chunks, body, 0)

    out = emb(idx3, table)
    return out.reshape(B0, S, D)


# speedup vs baseline: 9.1382x; 1.4379x over previous
import functools

import jax
import jax.numpy as jnp
from jax import lax
from jax.experimental import pallas as pl
from jax.experimental.pallas import tpu as pltpu
from jax.experimental.pallas import tpu_sc as plsc

NBUF = 4


def kernel(input_ids, table):
    B0, S = input_ids.shape
    V, D = table.shape
    B = B0 * S

    info = plsc.get_sparse_core_info()
    NC, NS = info.num_cores, info.num_subcores
    NW = NC * NS

    CHUNK = 128
    b_per_w = B // NW
    n_chunks = b_per_w // CHUNK
    n_groups = n_chunks // NBUF
    assert b_per_w * NW == B
    assert n_chunks * CHUNK == b_per_w
    assert n_groups * NBUF == n_chunks and n_groups >= 2

    idx3 = input_ids.reshape(NW, n_chunks, CHUNK).astype(jnp.int32)

    mesh = plsc.VectorSubcoreMesh(core_axis_name="c", subcore_axis_name="s")

    @functools.partial(
        pl.kernel,
        mesh=mesh,
        out_type=jax.ShapeDtypeStruct((B, D), jnp.float32),
        scratch_types=[
            pltpu.VMEM((n_chunks, CHUNK), jnp.int32),
            pltpu.VMEM((NBUF, CHUNK, D), jnp.float32),
        ]
        + [pltpu.SemaphoreType.DMA] * (2 * NBUF),
    )
    def emb(idx_hbm, table_hbm, out_hbm, idx_v, rows_v, *sems):
        gsem, osem = sems[:NBUF], sems[NBUF:]
        wid = lax.axis_index("s") * NC + lax.axis_index("c")
        base = wid * b_per_w
        pltpu.sync_copy(idx_hbm.at[wid], idx_v)

        def gather(j, b):
            return pltpu.make_async_copy(
                table_hbm.at[idx_v.at[j]], rows_v.at[b], gsem[b]
            )

        def store(j, b):
            return pltpu.make_async_copy(
                rows_v.at[b], out_hbm.at[pl.ds(base + j * CHUNK, CHUNK)], osem[b]
            )

        for b in range(NBUF):
            gather(b, b).start()

        def group(g, carry):
            for b in range(NBUF):
                j = g * NBUF + b
                gather(j, b).wait()
                store(j, b).start()
            for b in range(NBUF):
                j = g * NBUF + b
                store(j, b).wait()
                gather(j + NBUF, b).start()
            return carry

        lax.fori_loop(0, n_groups - 1, group, 0)

        gl = n_groups - 1
        for b in range(NBUF):
            j = gl * NBUF + b
            gather(j, b).wait()
            store(j, b).start()
        for b in range(NBUF):
            store(gl * NBUF + b, b).wait()

    out = emb(idx3, table)
    return out.reshape(B0, S, D)


# speedup vs baseline: 9.1926x; 1.0059x over previous
import functools

import jax
import jax.numpy as jnp
from jax import lax
from jax.experimental import pallas as pl
from jax.experimental.pallas import tpu as pltpu
from jax.experimental.pallas import tpu_sc as plsc

N = 2


def kernel(input_ids, table):
    B0, S = input_ids.shape
    V, D = table.shape
    B = B0 * S

    info = plsc.get_sparse_core_info()
    NC, NS = info.num_cores, info.num_subcores
    NW = NC * NS

    CHUNK = 128
    b_per_w = B // NW
    n_chunks = b_per_w // CHUNK
    n_super = n_chunks // (2 * N)
    assert b_per_w * NW == B
    assert n_chunks * CHUNK == b_per_w
    assert n_super * 2 * N == n_chunks and n_super >= 3

    idx3 = input_ids.reshape(NW, n_chunks, CHUNK).astype(jnp.int32)

    mesh = plsc.VectorSubcoreMesh(core_axis_name="c", subcore_axis_name="s")

    @functools.partial(
        pl.kernel,
        mesh=mesh,
        out_type=jax.ShapeDtypeStruct((B, D), jnp.float32),
        scratch_types=[
            pltpu.VMEM((n_chunks, CHUNK), jnp.int32),
            pltpu.VMEM((2 * N, CHUNK, D), jnp.float32),
        ]
        + [pltpu.SemaphoreType.DMA] * (4 * N),
    )
    def emb(idx_hbm, table_hbm, out_hbm, idx_v, rows_v, *sems):
        gsem, osem = sems[: 2 * N], sems[2 * N :]
        wid = lax.axis_index("s") * NC + lax.axis_index("c")
        base = wid * b_per_w
        pltpu.sync_copy(idx_hbm.at[wid], idx_v)

        def gather(j, buf):
            return pltpu.make_async_copy(
                table_hbm.at[idx_v.at[j]], rows_v.at[buf], gsem[buf]
            )

        def store(j, buf):
            return pltpu.make_async_copy(
                rows_v.at[buf], out_hbm.at[pl.ds(base + j * CHUNK, CHUNK)], osem[buf]
            )

        for b in range(N):
            gather(b, b).start()
        for b in range(N):
            gather(b, b).wait()
            store(b, b).start()
        for b in range(N):
            gather(N + b, N + b).start()
        for b in range(N):
            gather(N + b, N + b).wait()
            store(N + b, N + b).start()
        for b in range(N):
            store(b, b).wait()
            gather(2 * N + b, b).start()

        def body(sg, carry):
            j0 = sg * 2 * N
            for b in range(N):
                gather(j0 + b, b).wait()
                store(j0 + b, b).start()
            for b in range(N):
                store(j0 - N + b, N + b).wait()
                gather(j0 + N + b, N + b).start()
            for b in range(N):
                gather(j0 + N + b, N + b).wait()
                store(j0 + N + b, N + b).start()
            for b in range(N):
                store(j0 + b, b).wait()
                gather(j0 + 2 * N + b, b).start()
            return carry

        lax.fori_loop(1, n_super - 1, body, 0)

        j0 = (n_super - 1) * 2 * N
        for b in range(N):
            gather(j0 + b, b).wait()
            store(j0 + b, b).start()
        for b in range(N):
            store(j0 - N + b, N + b).wait()
            gather(j0 + N + b, N + b).start()
        for b in range(N):
            gather(j0 + N + b, N + b).wait()
            store(j0 + N + b, N + b).start()
        for b in range(N):
            store(j0 + b, b).wait()
        for b in range(N):
            store(j0 + N + b, N + b).wait()

    out = emb(idx3, table)
    return out.reshape(B0, S, D)
